# Initial kernel scaffold; baseline (speedup 1.0000x reference)
import functools

import jax
import jax.numpy as jnp
from jax import lax
from jax.experimental import pallas as pl
from jax.experimental.pallas import tpu as pltpu
from jax.experimental.pallas import tpu_sc as plsc

N_VOCAB = 1000000
D_EMB = 32

_info = plsc.get_sparse_core_info()
NC = _info.num_cores
NS = _info.num_subcores
NW = NC * NS

B_TOTAL = 4096 * 200
B_PER_W = B_TOTAL // NW
GROUP = 128
N_GROUPS = B_PER_W // GROUP
NBUF = 4


def _emb_kernel(table_hbm, idx_hbm, out_hbm, idx_v, rows_v, sems):
    wid = lax.axis_index("s") * NC + lax.axis_index("c")

    pltpu.sync_copy(idx_hbm.at[wid], idx_v)

    def fire(g, b):
        pltpu.async_copy(table_hbm.at[idx_v.at[g]], rows_v.at[b], sems.at[b])

    def wait_store(g, b):
        pltpu.make_async_copy(
            table_hbm.at[idx_v.at[g]], rows_v.at[b], sems.at[b]
        ).wait()
        pltpu.sync_copy(rows_v.at[b], out_hbm.at[wid, g])

    for b in range(NBUF):
        fire(b, b)

    @pl.loop(0, N_GROUPS - NBUF, step=NBUF)
    def _steady(i):
        for b in range(NBUF):
            g = i + b
            wait_store(g, b)
            fire(g + NBUF, b)

    for b in range(NBUF):
        wait_store(N_GROUPS - NBUF + b, b)


@jax.jit
def _emb(table, idx3):
    mesh = plsc.VectorSubcoreMesh(core_axis_name="c", subcore_axis_name="s")
    run = functools.partial(
        pl.kernel,
        out_type=jax.ShapeDtypeStruct((NW, N_GROUPS, GROUP, D_EMB), jnp.float32),
        mesh=mesh,
        scratch_types=[
            pltpu.VMEM((N_GROUPS, GROUP), jnp.int32),
            pltpu.VMEM((NBUF, GROUP, D_EMB), jnp.float32),
            pltpu.SemaphoreType.DMA((NBUF,)),
        ],
    )(_emb_kernel)
    return run(table, idx3)


def kernel(x, table):
    idx3 = x.reshape(NW, N_GROUPS, GROUP).astype(jnp.int32)
    out = _emb(table, idx3)
    return out.reshape(x.shape[0], x.shape[1], D_EMB)


# speedup vs baseline: 1.4915x; 1.4915x over previous
import functools

import jax
import jax.numpy as jnp
from jax import lax
from jax.experimental import pallas as pl
from jax.experimental.pallas import tpu as pltpu
from jax.experimental.pallas import tpu_sc as plsc

N_VOCAB = 1000000
D_EMB = 32

_info = plsc.get_sparse_core_info()
NC = _info.num_cores
NS = _info.num_subcores
NW = NC * NS

B_TOTAL = 4096 * 200
B_PER_W = B_TOTAL // NW
GROUP = 128
N_GROUPS = B_PER_W // GROUP
NBUF = 4


def _emb_kernel(table_hbm, idx_hbm, out_hbm, idx_v, rows_v, sems):
    wid = lax.axis_index("s") * NC + lax.axis_index("c")

    pltpu.sync_copy(idx_hbm.at[wid], idx_v)

    def fire(g, b):
        pltpu.async_copy(table_hbm.at[idx_v.at[g]], rows_v.at[b], sems.at[b])

    def wait_store(g, b):
        pltpu.make_async_copy(
            table_hbm.at[idx_v.at[g]], rows_v.at[b], sems.at[b]
        ).wait()
        pltpu.sync_copy(rows_v.at[b], out_hbm.at[wid, g])

    for b in range(NBUF):
        fire(b, b)

    @pl.loop(0, N_GROUPS - NBUF, step=NBUF)
    def _steady(i):
        for b in range(NBUF):
            g = i + b
            wait_store(g, b)
            fire(g + NBUF, b)

    for b in range(NBUF):
        wait_store(N_GROUPS - NBUF + b, b)


@jax.jit
def _emb(table, idx3):
    mesh = plsc.VectorSubcoreMesh(core_axis_name="c", subcore_axis_name="s")
    run = functools.partial(
        pl.kernel,
        out_type=jax.ShapeDtypeStruct((NW, N_GROUPS, GROUP, D_EMB), jnp.float32),
        mesh=mesh,
        scratch_types=[
            pltpu.VMEM((N_GROUPS, GROUP), jnp.int32),
            pltpu.VMEM((NBUF, GROUP, D_EMB), jnp.float32),
            pltpu.SemaphoreType.DMA((NBUF,)),
        ],
        compiler_params=pltpu.CompilerParams(use_tc_tiling_on_sc=False),
    )(_emb_kernel)
    return run(table, idx3)


def kernel(x, table):
    idx3 = x.reshape(NW, N_GROUPS, GROUP).astype(jnp.int32)
    out = _emb(table, idx3)
    return out.reshape(x.shape[0], x.shape[1], D_EMB)


# speedup vs baseline: 1.5011x; 1.0064x over previous
import functools

import jax
import jax.numpy as jnp
from jax import lax
from jax.experimental import pallas as pl
from jax.experimental.pallas import tpu as pltpu
from jax.experimental.pallas import tpu_sc as plsc

N_VOCAB = 1000000
D_EMB = 32

_info = plsc.get_sparse_core_info()
NC = _info.num_cores
NS = _info.num_subcores
NW = NC * NS

B_TOTAL = 4096 * 200
B_PER_W = B_TOTAL // NW
GROUP = 128
N_GROUPS = B_PER_W // GROUP
NBUF = 12
LOOKAHEAD = 10


def _emb_kernel(table_hbm, idx_hbm, out_hbm, idx_v, rows_v, gsems, ssems):
    wid = lax.axis_index("s") * NC + lax.axis_index("c")

    pltpu.sync_copy(idx_hbm.at[wid], idx_v)

    def fire_g(g, b):
        pltpu.async_copy(table_hbm.at[idx_v.at[g]], rows_v.at[b], gsems.at[b])

    def wait_g(g, b):
        pltpu.make_async_copy(
            table_hbm.at[idx_v.at[g]], rows_v.at[b], gsems.at[b]
        ).wait()

    def fire_s(g, b):
        pltpu.async_copy(rows_v.at[b], out_hbm.at[wid, g], ssems.at[b])

    def wait_s(g, b):
        pltpu.make_async_copy(
            rows_v.at[b], out_hbm.at[wid, g], ssems.at[b]
        ).wait()

    def step(g, b, *, do_wait_s, do_fire_g):
        wait_g(g, b)
        fire_s(g, b)
        if do_fire_g:
            bf = (g + LOOKAHEAD) % NBUF if isinstance(g, int) else b
            if do_wait_s:
                wait_s(g + LOOKAHEAD - NBUF, bf)
            fire_g(g + LOOKAHEAD, bf)

    for g in range(LOOKAHEAD):
        fire_g(g, g % NBUF)

    for g in range(NBUF):
        step(g, g % NBUF,
             do_wait_s=(g + LOOKAHEAD - NBUF >= 0),
             do_fire_g=(g + LOOKAHEAD < N_GROUPS))

    @pl.loop(NBUF, N_GROUPS - 20, step=NBUF)
    def _main(i):
        for b in range(NBUF):
            g = i + b
            wait_g(g, b)
            fire_s(g, b)
            bf = (b + LOOKAHEAD) % NBUF
            wait_s(g + LOOKAHEAD - NBUF, bf)
            fire_g(g + LOOKAHEAD, bf)

    for g in range(N_GROUPS - 20, N_GROUPS):
        step(g, g % NBUF,
             do_wait_s=True,
             do_fire_g=(g + LOOKAHEAD < N_GROUPS))

    for g in range(N_GROUPS - NBUF, N_GROUPS):
        wait_s(g, g % NBUF)


@jax.jit
def _emb(table, idx3):
    mesh = plsc.VectorSubcoreMesh(core_axis_name="c", subcore_axis_name="s")
    run = functools.partial(
        pl.kernel,
        out_type=jax.ShapeDtypeStruct((NW, N_GROUPS, GROUP, D_EMB), jnp.float32),
        mesh=mesh,
        scratch_types=[
            pltpu.VMEM((N_GROUPS, GROUP), jnp.int32),
            pltpu.VMEM((NBUF, GROUP, D_EMB), jnp.float32),
            pltpu.SemaphoreType.DMA((NBUF,)),
            pltpu.SemaphoreType.DMA((NBUF,)),
        ],
        compiler_params=pltpu.CompilerParams(use_tc_tiling_on_sc=False),
    )(_emb_kernel)
    return run(table, idx3)


def kernel(x, table):
    idx3 = x.reshape(NW, N_GROUPS, GROUP).astype(jnp.int32)
    out = _emb(table, idx3)
    return out.reshape(x.shape[0], x.shape[1], D_EMB)
